# baseline (speedup 1.0000x reference)
import functools

import jax
import jax.numpy as jnp
from jax import lax
from jax.experimental import pallas as pl
from jax.experimental.pallas import tpu as pltpu
from jax.experimental.pallas import tpu_sc as plsc

ROWS, COLS = 128, 32768
L = 16
NUM_CORES = 2
NUM_WORKERS = 32
RPW = ROWS // NUM_WORKERS
CHUNK = 8192
CPR = COLS // CHUNK
SEGC = CHUNK // L
MAIN = SEGC - 16
SEGMAX = SEGC + 1
NT = RPW * CPR
NBUF = 4

_mesh = plsc.VectorSubcoreMesh(core_axis_name="c", subcore_axis_name="s")


@functools.partial(
    pl.kernel,
    out_type=jax.ShapeDtypeStruct((ROWS * COLS,), jnp.float32),
    mesh=_mesh,
    scratch_types=[
        [pltpu.VMEM((CHUNK,), jnp.float32)] * NBUF,
        [pltpu.VMEM((CHUNK,), jnp.float32)] * NBUF,
        [pltpu.SemaphoreType.DMA] * NBUF,
        [pltpu.SemaphoreType.DMA] * NBUF,
    ],
    compiler_params=pltpu.CompilerParams(needs_layout_passes=False),
)
def _scan_rows(x_hbm, out_hbm, inb, outb, sin, sout):
    wid = lax.axis_index("s") * NUM_CORES + lax.axis_index("c")
    iota = lax.iota(jnp.int32, L)
    startv = iota * SEGC + iota
    lenv = jnp.where(iota < L - 1, SEGC + 1, SEGC - (L - 1))

    def hbm_off(t):
        row = wid * RPW + t // CPR
        return row * COLS + (t % CPR) * CHUNK

    loads = [None] * NT
    stores = [None] * NT
    for t0 in range(NBUF - 1):
        loads[t0] = pltpu.async_copy(
            x_hbm.at[pl.ds(hbm_off(t0), CHUNK)], inb[t0], sin[t0]
        )

    row_carry = jnp.float32(0)
    for t in range(NT):
        s = t % NBUF
        loads[t].wait()
        if t + NBUF - 1 < NT:
            tn = t + NBUF - 1
            loads[tn] = pltpu.async_copy(
                x_hbm.at[pl.ds(hbm_off(tn), CHUNK)], inb[tn % NBUF],
                sin[tn % NBUF],
            )
        if t % CPR == 0:
            row_carry = jnp.float32(0)

        ib, ob = inb[s], outb[s]
        z = jnp.zeros((L,), jnp.float32)

        @plsc.parallel_loop(0, MAIN, step=4, unroll=4, carry=(z, z, z, z))
        def _pass_a(k, accs):
            a0, a1, a2, a3 = accs
            a0 = a0 + plsc.load_gather(ib, [startv + k])
            a1 = a1 + plsc.load_gather(ib, [startv + (k + 1)])
            a2 = a2 + plsc.load_gather(ib, [startv + (k + 2)])
            a3 = a3 + plsc.load_gather(ib, [startv + (k + 3)])
            return a0, a1, a2, a3

        a0, a1, a2, a3 = _pass_a

        def _tail_a(k, acc):
            m = k < lenv
            v = plsc.load_gather(ib, [startv + k], mask=m)
            return acc + jnp.where(m, v, 0.0)

        at = lax.fori_loop(MAIN, SEGMAX, _tail_a, z)
        seg_sums = ((a0 + a1) + (a2 + a3)) + at
        inc = plsc.cumsum(seg_sums)
        lane_base = (inc - seg_sums) + row_carry
        row_carry = row_carry + jnp.sum(seg_sums)

        if t >= NBUF:
            stores[t - NBUF].wait()

        @plsc.parallel_loop(0, MAIN, step=4, unroll=4, carry=lane_base)
        def _pass_b(k, carry):
            v0 = plsc.load_gather(ib, [startv + k])
            v1 = plsc.load_gather(ib, [startv + (k + 1)])
            v2 = plsc.load_gather(ib, [startv + (k + 2)])
            v3 = plsc.load_gather(ib, [startv + (k + 3)])
            plsc.store_scatter(ob, [startv + k], carry)
            c1 = carry + v0
            plsc.store_scatter(ob, [startv + (k + 1)], c1)
            c2 = c1 + v1
            plsc.store_scatter(ob, [startv + (k + 2)], c2)
            c3 = c2 + v2
            plsc.store_scatter(ob, [startv + (k + 3)], c3)
            return c3 + v3

        def _tail_b(k, carry):
            m = k < lenv
            v = plsc.load_gather(ib, [startv + k], mask=m)
            plsc.store_scatter(ob, [startv + k], carry, mask=m)
            return carry + jnp.where(m, v, 0.0)

        lax.fori_loop(MAIN, SEGMAX, _tail_b, _pass_b)
        stores[t] = pltpu.async_copy(
            ob, out_hbm.at[pl.ds(hbm_off(t), CHUNK)], sout[s]
        )

    for t in range(max(NT - NBUF, 0), NT):
        stores[t].wait()


def kernel(x):
    return _scan_rows(x.reshape(-1)).reshape(ROWS, COLS)


# speedup vs baseline: 1.8423x; 1.8423x over previous
import functools

import jax
import jax.numpy as jnp
from jax import lax
from jax.experimental import pallas as pl
from jax.experimental.pallas import tpu as pltpu
from jax.experimental.pallas import tpu_sc as plsc

ROWS, COLS = 128, 32768
L = 16
NUM_CORES = 2
NUM_WORKERS = 32
RPW = ROWS // NUM_WORKERS
CHUNK = 8192
CPR = COLS // CHUNK
SEGC = CHUNK // L
MAIN = SEGC - 16
SEGMAX = SEGC + 1
NT = RPW * CPR
NBUF = 4

_mesh = plsc.VectorSubcoreMesh(core_axis_name="c", subcore_axis_name="s")


@functools.partial(
    pl.kernel,
    out_type=jax.ShapeDtypeStruct((ROWS, COLS), jnp.float32),
    mesh=_mesh,
    scratch_types=[
        [pltpu.VMEM((CHUNK,), jnp.float32)] * NBUF,
        [pltpu.VMEM((CHUNK,), jnp.float32)] * NBUF,
        [pltpu.SemaphoreType.DMA] * NBUF,
        [pltpu.SemaphoreType.DMA] * NBUF,
    ],
    compiler_params=pltpu.CompilerParams(needs_layout_passes=False),
)
def _scan_rows(x_hbm, out_hbm, inb, outb, sin, sout):
    wid = lax.axis_index("s") * NUM_CORES + lax.axis_index("c")
    iota = lax.iota(jnp.int32, L)
    startv = iota * SEGC + iota
    lenv = jnp.where(iota < L - 1, SEGC + 1, SEGC - (L - 1))

    def hbm_slice(ref, t):
        row = wid * RPW + t // CPR
        return ref.at[row, pl.ds((t % CPR) * CHUNK, CHUNK)]

    loads = [None] * NT
    stores = [None] * NT
    for t0 in range(NBUF - 1):
        loads[t0] = pltpu.async_copy(
            hbm_slice(x_hbm, t0), inb[t0], sin[t0]
        )

    row_carry = jnp.float32(0)
    for t in range(NT):
        s = t % NBUF
        loads[t].wait()
        if t + NBUF - 1 < NT:
            tn = t + NBUF - 1
            loads[tn] = pltpu.async_copy(
                hbm_slice(x_hbm, tn), inb[tn % NBUF],
                sin[tn % NBUF],
            )
        if t % CPR == 0:
            row_carry = jnp.float32(0)

        ib, ob = inb[s], outb[s]
        z = jnp.zeros((L,), jnp.float32)

        @plsc.parallel_loop(0, MAIN, step=4, unroll=4, carry=(z, z, z, z))
        def _pass_a(k, accs):
            a0, a1, a2, a3 = accs
            a0 = a0 + plsc.load_gather(ib, [startv + k])
            a1 = a1 + plsc.load_gather(ib, [startv + (k + 1)])
            a2 = a2 + plsc.load_gather(ib, [startv + (k + 2)])
            a3 = a3 + plsc.load_gather(ib, [startv + (k + 3)])
            return a0, a1, a2, a3

        a0, a1, a2, a3 = _pass_a

        def _tail_a(k, acc):
            m = k < lenv
            v = plsc.load_gather(ib, [startv + k], mask=m)
            return acc + jnp.where(m, v, 0.0)

        at = lax.fori_loop(MAIN, SEGMAX, _tail_a, z)
        seg_sums = ((a0 + a1) + (a2 + a3)) + at
        inc = plsc.cumsum(seg_sums)
        lane_base = (inc - seg_sums) + row_carry
        row_carry = row_carry + jnp.sum(seg_sums)

        if t >= NBUF:
            stores[t - NBUF].wait()

        @plsc.parallel_loop(0, MAIN, step=4, unroll=4, carry=lane_base)
        def _pass_b(k, carry):
            v0 = plsc.load_gather(ib, [startv + k])
            v1 = plsc.load_gather(ib, [startv + (k + 1)])
            v2 = plsc.load_gather(ib, [startv + (k + 2)])
            v3 = plsc.load_gather(ib, [startv + (k + 3)])
            plsc.store_scatter(ob, [startv + k], carry)
            c1 = carry + v0
            plsc.store_scatter(ob, [startv + (k + 1)], c1)
            c2 = c1 + v1
            plsc.store_scatter(ob, [startv + (k + 2)], c2)
            c3 = c2 + v2
            plsc.store_scatter(ob, [startv + (k + 3)], c3)
            return c3 + v3

        def _tail_b(k, carry):
            m = k < lenv
            v = plsc.load_gather(ib, [startv + k], mask=m)
            plsc.store_scatter(ob, [startv + k], carry, mask=m)
            return carry + jnp.where(m, v, 0.0)

        lax.fori_loop(MAIN, SEGMAX, _tail_b, _pass_b)
        stores[t] = pltpu.async_copy(
            ob, hbm_slice(out_hbm, t), sout[s]
        )

    for t in range(max(NT - NBUF, 0), NT):
        stores[t].wait()


def kernel(x):
    return _scan_rows(x)


# speedup vs baseline: 2.0375x; 1.1060x over previous
import functools

import jax
import jax.numpy as jnp
from jax import lax
from jax.experimental import pallas as pl
from jax.experimental.pallas import tpu as pltpu
from jax.experimental.pallas import tpu_sc as plsc

ROWS, COLS = 128, 32768
L = 16
NUM_CORES = 2
NUM_WORKERS = 32
RPW = ROWS // NUM_WORKERS
CHUNK = 8192
CPR = COLS // CHUNK
SEGC = CHUNK // L
MAIN = SEGC - 16
SEGMAX = SEGC + 1
NT = RPW * CPR
NBUF = 4

_mesh = plsc.VectorSubcoreMesh(core_axis_name="c", subcore_axis_name="s")


@functools.partial(
    pl.kernel,
    out_type=jax.ShapeDtypeStruct((ROWS, COLS), jnp.float32),
    mesh=_mesh,
    scratch_types=[
        [pltpu.VMEM((CHUNK,), jnp.float32)] * NBUF,
        [pltpu.VMEM((CHUNK,), jnp.float32)] * NBUF,
        [pltpu.SemaphoreType.DMA] * NBUF,
        [pltpu.SemaphoreType.DMA] * NBUF,
    ],
    compiler_params=pltpu.CompilerParams(needs_layout_passes=False),
)
def _scan_rows(x_hbm, out_hbm, inb, outb, sin, sout):
    wid = lax.axis_index("s") * NUM_CORES + lax.axis_index("c")
    iota = lax.iota(jnp.int32, L)
    startv = iota * SEGC + iota
    lenv = jnp.where(iota < L - 1, SEGC + 1, SEGC - (L - 1))

    def hbm_slice(ref, t):
        row = wid * RPW + t // CPR
        return ref.at[row, pl.ds((t % CPR) * CHUNK, CHUNK)]

    loads = [None] * NT
    stores = [None] * NT
    for t0 in range(NBUF - 1):
        loads[t0] = pltpu.async_copy(
            hbm_slice(x_hbm, t0), inb[t0], sin[t0]
        )

    row_carry = jnp.float32(0)
    for t in range(NT):
        s = t % NBUF
        loads[t].wait()
        if t + NBUF - 1 < NT:
            tn = t + NBUF - 1
            loads[tn] = pltpu.async_copy(
                hbm_slice(x_hbm, tn), inb[tn % NBUF],
                sin[tn % NBUF],
            )
        if t % CPR == 0:
            row_carry = jnp.float32(0)

        ib, ob = inb[s], outb[s]
        z = jnp.zeros((L,), jnp.float32)

        @plsc.parallel_loop(0, MAIN, step=4, unroll=4, carry=(z, z, z, z))
        def _pass_a(k, accs):
            a0, a1, a2, a3 = accs
            a0 = a0 + plsc.load_gather(ib, [startv + k])
            a1 = a1 + plsc.load_gather(ib, [startv + (k + 1)])
            a2 = a2 + plsc.load_gather(ib, [startv + (k + 2)])
            a3 = a3 + plsc.load_gather(ib, [startv + (k + 3)])
            return a0, a1, a2, a3

        a0, a1, a2, a3 = _pass_a

        def _tail_a(k, acc):
            m = k < lenv
            v = plsc.load_gather(ib, [startv + k], mask=m)
            return acc + jnp.where(m, v, 0.0)

        at = lax.fori_loop(MAIN, SEGMAX, _tail_a, z)
        seg_sums = ((a0 + a1) + (a2 + a3)) + at
        inc = plsc.cumsum(seg_sums)
        lane_base = (inc - seg_sums) + row_carry
        row_carry = row_carry + jnp.sum(seg_sums)

        if t >= NBUF:
            stores[t - NBUF].wait()

        @plsc.parallel_loop(0, MAIN, step=4, unroll=4, carry=lane_base)
        def _pass_b(k, carry):
            v0 = plsc.load_gather(ib, [startv + k])
            v1 = plsc.load_gather(ib, [startv + (k + 1)])
            v2 = plsc.load_gather(ib, [startv + (k + 2)])
            v3 = plsc.load_gather(ib, [startv + (k + 3)])
            s01 = v0 + v1
            s012 = s01 + v2
            s0123 = s012 + v3
            plsc.store_scatter(ob, [startv + k], carry)
            plsc.store_scatter(ob, [startv + (k + 1)], carry + v0)
            plsc.store_scatter(ob, [startv + (k + 2)], carry + s01)
            plsc.store_scatter(ob, [startv + (k + 3)], carry + s012)
            return carry + s0123

        def _tail_b(k, carry):
            m = k < lenv
            v = plsc.load_gather(ib, [startv + k], mask=m)
            plsc.store_scatter(ob, [startv + k], carry, mask=m)
            return carry + jnp.where(m, v, 0.0)

        lax.fori_loop(MAIN, SEGMAX, _tail_b, _pass_b)
        stores[t] = pltpu.async_copy(
            ob, hbm_slice(out_hbm, t), sout[s]
        )

    for t in range(max(NT - NBUF, 0), NT):
        stores[t].wait()


def kernel(x):
    return _scan_rows(x)


# speedup vs baseline: 2.1537x; 1.0570x over previous
import functools

import jax
import jax.numpy as jnp
from jax import lax
from jax.experimental import pallas as pl
from jax.experimental.pallas import tpu as pltpu
from jax.experimental.pallas import tpu_sc as plsc

ROWS, COLS = 128, 32768
L = 16
NUM_CORES = 2
NUM_WORKERS = 32
RPW = ROWS // NUM_WORKERS
CHUNK = 16384
CPR = COLS // CHUNK
SEGC = CHUNK // L
MAIN = SEGC - 16
SEGMAX = SEGC + 1
NT = RPW * CPR
NBUF = 2

_mesh = plsc.VectorSubcoreMesh(core_axis_name="c", subcore_axis_name="s")


@functools.partial(
    pl.kernel,
    out_type=jax.ShapeDtypeStruct((ROWS, COLS), jnp.float32),
    mesh=_mesh,
    scratch_types=[
        [pltpu.VMEM((CHUNK,), jnp.float32)] * NBUF,
        [pltpu.VMEM((CHUNK,), jnp.float32)] * NBUF,
        [pltpu.SemaphoreType.DMA] * NBUF,
        [pltpu.SemaphoreType.DMA] * NBUF,
    ],
    compiler_params=pltpu.CompilerParams(needs_layout_passes=False),
)
def _scan_rows(x_hbm, out_hbm, inb, outb, sin, sout):
    wid = lax.axis_index("s") * NUM_CORES + lax.axis_index("c")
    iota = lax.iota(jnp.int32, L)
    startv = iota * SEGC + iota
    lenv = jnp.where(iota < L - 1, SEGC + 1, SEGC - (L - 1))

    def hbm_slice(ref, t):
        row = wid * RPW + t // CPR
        return ref.at[row, pl.ds((t % CPR) * CHUNK, CHUNK)]

    loads = [None] * NT
    stores = [None] * NT
    for t0 in range(NBUF - 1):
        loads[t0] = pltpu.async_copy(
            hbm_slice(x_hbm, t0), inb[t0], sin[t0]
        )

    row_carry = jnp.float32(0)
    for t in range(NT):
        s = t % NBUF
        loads[t].wait()
        if t + NBUF - 1 < NT:
            tn = t + NBUF - 1
            loads[tn] = pltpu.async_copy(
                hbm_slice(x_hbm, tn), inb[tn % NBUF],
                sin[tn % NBUF],
            )
        if t % CPR == 0:
            row_carry = jnp.float32(0)

        ib, ob = inb[s], outb[s]
        z = jnp.zeros((L,), jnp.float32)

        @plsc.parallel_loop(0, MAIN, step=4, unroll=4, carry=(z, z, z, z))
        def _pass_a(k, accs):
            a0, a1, a2, a3 = accs
            a0 = a0 + plsc.load_gather(ib, [startv + k])
            a1 = a1 + plsc.load_gather(ib, [startv + (k + 1)])
            a2 = a2 + plsc.load_gather(ib, [startv + (k + 2)])
            a3 = a3 + plsc.load_gather(ib, [startv + (k + 3)])
            return a0, a1, a2, a3

        a0, a1, a2, a3 = _pass_a

        def _tail_a(k, acc):
            m = k < lenv
            v = plsc.load_gather(ib, [startv + k], mask=m)
            return acc + jnp.where(m, v, 0.0)

        at = lax.fori_loop(MAIN, SEGMAX, _tail_a, z)
        seg_sums = ((a0 + a1) + (a2 + a3)) + at
        inc = plsc.cumsum(seg_sums)
        lane_base = (inc - seg_sums) + row_carry
        row_carry = row_carry + jnp.sum(seg_sums)

        if t >= NBUF:
            stores[t - NBUF].wait()

        @plsc.parallel_loop(0, MAIN, step=4, unroll=4, carry=lane_base)
        def _pass_b(k, carry):
            v0 = plsc.load_gather(ib, [startv + k])
            v1 = plsc.load_gather(ib, [startv + (k + 1)])
            v2 = plsc.load_gather(ib, [startv + (k + 2)])
            v3 = plsc.load_gather(ib, [startv + (k + 3)])
            s01 = v0 + v1
            s012 = s01 + v2
            s0123 = s012 + v3
            plsc.store_scatter(ob, [startv + k], carry)
            plsc.store_scatter(ob, [startv + (k + 1)], carry + v0)
            plsc.store_scatter(ob, [startv + (k + 2)], carry + s01)
            plsc.store_scatter(ob, [startv + (k + 3)], carry + s012)
            return carry + s0123

        def _tail_b(k, carry):
            m = k < lenv
            v = plsc.load_gather(ib, [startv + k], mask=m)
            plsc.store_scatter(ob, [startv + k], carry, mask=m)
            return carry + jnp.where(m, v, 0.0)

        lax.fori_loop(MAIN, SEGMAX, _tail_b, _pass_b)
        stores[t] = pltpu.async_copy(
            ob, hbm_slice(out_hbm, t), sout[s]
        )

    for t in range(max(NT - NBUF, 0), NT):
        stores[t].wait()


def kernel(x):
    return _scan_rows(x)
